# Initial kernel scaffold; baseline (speedup 1.0000x reference)
import jax
import jax.numpy as jnp
from jax import lax
from jax.experimental import pallas as pl

_K = 512
_C = 64
_S = 5
_N = 16384
_BN = 256


def _prep_kernel(d_ref, dn_ref, gram_ref):
    d = d_ref[...]
    norm = jnp.sqrt(jnp.sum(d * d, axis=0, keepdims=True))
    dn = d / norm
    dn_ref[...] = dn
    gram_ref[...] = lax.dot_general(
        dn, dn, (((0,), (0,)), ((), ())), preferred_element_type=jnp.float32)


def _omp_kernel(s_ref, dn_ref, gram_ref, coeff_ref, zdl_ref, loss_ref):
    s = s_ref[...]
    dn = dn_ref[...]
    gram = gram_ref[...]
    init_corr = lax.dot_general(
        dn, s, (((0,), (0,)), ((), ())), preferred_element_type=jnp.float32)

    iota = lax.broadcasted_iota(jnp.int32, (_K, _BN), 0)
    corr = init_corr
    omega = jnp.ones((_K, _BN), jnp.float32)
    L = {}
    alphas = []
    g_cols = []
    sel_idx = []
    coeffs = None

    for kk in range(_S):
        a = jnp.abs(corr) * omega
        m = jnp.max(a, axis=0, keepdims=True)
        idx = jnp.min(jnp.where(a == m, iota, _K), axis=0, keepdims=True)
        onehot = (iota == idx).astype(jnp.float32)
        omega = omega * (1.0 - onehot)
        g = jnp.dot(gram, onehot, preferred_element_type=jnp.float32)
        alphas.append(jnp.sum(init_corr * onehot, axis=0, keepdims=True))
        if kk > 0:
            gent = [jnp.sum(g_cols[j] * onehot, axis=0, keepdims=True)
                    for j in range(kk)]
            w = []
            for j in range(kk):
                acc = gent[j]
                for c in range(j):
                    acc = acc - L[(j, c)] * w[c]
                w.append(acc / L[(j, j)])
            wsq = w[0] * w[0]
            for j in range(1, kk):
                wsq = wsq + w[j] * w[j]
            for c in range(kk):
                L[(kk, c)] = w[c]
            L[(kk, kk)] = jnp.sqrt(1.0 - wsq)
        else:
            L[(0, 0)] = jnp.ones((1, _BN), jnp.float32)
        g_cols.append(g)
        sel_idx.append(idx)

        n = kk + 1
        y = []
        for j in range(n):
            acc = alphas[j]
            for c in range(j):
                acc = acc - L[(j, c)] * y[c]
            y.append(acc / L[(j, j)])
        cvec = [None] * n
        for j in reversed(range(n)):
            acc = y[j]
            for r in range(j + 1, n):
                acc = acc - L[(r, j)] * cvec[r]
            cvec[j] = acc / L[(j, j)]
        coeffs = cvec

        if kk < _S - 1:
            beta = cvec[0] * g_cols[0]
            for j in range(1, n):
                beta = beta + cvec[j] * g_cols[j]
            corr = init_corr - beta

    cd = coeffs[0] * (iota == sel_idx[0]).astype(jnp.float32)
    for j in range(1, _S):
        cd = cd + coeffs[j] * (iota == sel_idx[j]).astype(jnp.float32)
    coeff_ref[...] = cd

    zdl = jnp.dot(dn, cd, preferred_element_type=jnp.float32)
    zdl_ref[...] = zdl
    diff = zdl - s
    sq = jnp.sum(diff * diff)

    i = pl.program_id(0)

    @pl.when(i == 0)
    def _init():
        loss_ref[0, 0] = 0.0

    loss_ref[0, 0] += sq


def kernel(z_e, dictionary):
    z = jnp.transpose(z_e, (0, 2, 3, 1))
    input_shape = z.shape
    s = z.reshape(_C, -1)

    dn, gram = pl.pallas_call(
        _prep_kernel,
        out_shape=(
            jax.ShapeDtypeStruct((_C, _K), jnp.float32),
            jax.ShapeDtypeStruct((_K, _K), jnp.float32),
        ),
    )(dictionary)

    grid = _N // _BN
    coeff, zdl, loss_sum = pl.pallas_call(
        _omp_kernel,
        grid=(grid,),
        in_specs=[
            pl.BlockSpec((_C, _BN), lambda i: (0, i)),
            pl.BlockSpec((_C, _K), lambda i: (0, 0)),
            pl.BlockSpec((_K, _K), lambda i: (0, 0)),
        ],
        out_specs=(
            pl.BlockSpec((_K, _BN), lambda i: (0, i)),
            pl.BlockSpec((_C, _BN), lambda i: (0, i)),
            pl.BlockSpec((1, 1), lambda i: (0, 0)),
        ),
        out_shape=(
            jax.ShapeDtypeStruct((_K, _N), jnp.float32),
            jax.ShapeDtypeStruct((_C, _N), jnp.float32),
            jax.ShapeDtypeStruct((1, 1), jnp.float32),
        ),
    )(s, dn, gram)

    mse = loss_sum[0, 0] / (input_shape[0] * input_shape[1] * input_shape[2] * input_shape[3])
    loss = 0.25 * mse + mse
    out = jnp.transpose(zdl.reshape(input_shape), (0, 3, 1, 2))
    return out, loss, coeff


# speedup vs baseline: 6.8065x; 6.8065x over previous
import jax
import jax.numpy as jnp
from jax import lax
from jax.experimental import pallas as pl

_K = 512
_C = 64
_S = 5
_N = 16384
_BN = 256


def _bf(x):
    u = lax.bitcast_convert_type(x, jnp.int32)
    lsb = lax.shift_right_logical(u, 16) & 1
    u = u + (32767 + lsb)
    u = u & jnp.int32(-65536)
    return lax.bitcast_convert_type(u, jnp.float32)


def _prep_kernel(dn_ref, gram_ref):
    dn = dn_ref[...]
    gram_ref[...] = lax.dot_general(
        _bf(dn), _bf(dn), (((0,), (0,)), ((), ())),
        preferred_element_type=jnp.float32)


def _omp_kernel(s_ref, dn_ref, gram_ref, coeff_ref, zdl_ref, loss_ref):
    s = s_ref[...]
    dn = dn_ref[...]
    gram = gram_ref[...]
    init_corr = lax.dot_general(
        _bf(dn), _bf(s), (((0,), (0,)), ((), ())),
        preferred_element_type=jnp.float32)

    iota = lax.broadcasted_iota(jnp.int32, (_K, _BN), 0)
    corr = init_corr
    omega = jnp.ones((_K, _BN), jnp.float32)
    L = {}
    alphas = []
    g_cols = []
    sel_idx = []
    coeffs = None

    for kk in range(_S):
        a = jnp.abs(corr) * omega
        m = jnp.max(a, axis=0, keepdims=True)
        idx = jnp.min(jnp.where(a == m, iota, _K), axis=0, keepdims=True)
        onehot = (iota == idx).astype(jnp.float32)
        omega = omega * (1.0 - onehot)
        g = jnp.dot(gram, onehot, precision=lax.Precision.HIGHEST,
                    preferred_element_type=jnp.float32)
        alphas.append(jnp.sum(init_corr * onehot, axis=0, keepdims=True))
        if kk > 0:
            gent = [jnp.sum(g_cols[j] * onehot, axis=0, keepdims=True)
                    for j in range(kk)]
            w = []
            for j in range(kk):
                acc = gent[j]
                for c in range(j):
                    acc = acc - L[(j, c)] * w[c]
                w.append(acc / L[(j, j)])
            wsq = w[0] * w[0]
            for j in range(1, kk):
                wsq = wsq + w[j] * w[j]
            for c in range(kk):
                L[(kk, c)] = w[c]
            L[(kk, kk)] = jnp.sqrt(1.0 - wsq)
        else:
            L[(0, 0)] = jnp.ones((1, _BN), jnp.float32)
        g_cols.append(g)
        sel_idx.append(idx)

        n = kk + 1
        mi = {}
        for i in range(n):
            mi[(i, i)] = 1.0 / L[(i, i)]
        for i in range(n):
            for j in range(i - 1, -1, -1):
                acc = jnp.zeros_like(L[(0, 0)])
                for mm in range(j, i):
                    acc = acc + L[(i, mm)] * mi[(mm, j)]
                mi[(i, j)] = -acc / L[(i, i)]
        y = []
        for i in range(n):
            acc = jnp.zeros_like(alphas[0])
            for j in range(i + 1):
                acc = acc + mi[(i, j)] * alphas[j]
            y.append(acc)
        cvec = [None] * n
        for i in range(n):
            acc = jnp.zeros_like(y[0])
            for j in range(i, n):
                acc = acc + mi[(j, i)] * y[j]
            cvec[i] = acc
        coeffs = cvec

        if kk < _S - 1:
            beta = cvec[0] * g_cols[0]
            for j in range(1, n):
                beta = beta + cvec[j] * g_cols[j]
            corr = init_corr - beta

    cd = coeffs[0] * (iota == sel_idx[0]).astype(jnp.float32)
    for j in range(1, _S):
        cd = cd + coeffs[j] * (iota == sel_idx[j]).astype(jnp.float32)
    coeff_ref[...] = cd

    zdl = lax.dot_general(_bf(dn), _bf(cd), (((1,), (0,)), ((), ())),
                          preferred_element_type=jnp.float32)
    zdl_ref[...] = zdl
    diff = zdl - s
    sq = jnp.sum(jnp.sum(diff * diff, axis=0, keepdims=True), axis=1, keepdims=True)

    i = pl.program_id(0)

    @pl.when(i == 0)
    def _init():
        loss_ref[...] = jnp.zeros((1, 1), jnp.float32)

    loss_ref[...] += sq


def kernel(z_e, dictionary):
    z = jnp.transpose(z_e, (0, 2, 3, 1))
    input_shape = z.shape
    s = z.reshape(_C, -1)
    dn = dictionary / jnp.linalg.norm(dictionary, axis=0)

    gram = pl.pallas_call(
        _prep_kernel,
        out_shape=jax.ShapeDtypeStruct((_K, _K), jnp.float32),
    )(dn)

    grid = _N // _BN
    coeff, zdl, loss_sum = pl.pallas_call(
        _omp_kernel,
        grid=(grid,),
        in_specs=[
            pl.BlockSpec((_C, _BN), lambda i: (0, i)),
            pl.BlockSpec((_C, _K), lambda i: (0, 0)),
            pl.BlockSpec((_K, _K), lambda i: (0, 0)),
        ],
        out_specs=(
            pl.BlockSpec((_K, _BN), lambda i: (0, i)),
            pl.BlockSpec((_C, _BN), lambda i: (0, i)),
            pl.BlockSpec((1, 1), lambda i: (0, 0)),
        ),
        out_shape=(
            jax.ShapeDtypeStruct((_K, _N), jnp.float32),
            jax.ShapeDtypeStruct((_C, _N), jnp.float32),
            jax.ShapeDtypeStruct((1, 1), jnp.float32),
        ),
    )(s, dn, gram)

    mse = loss_sum[0, 0] / (input_shape[0] * input_shape[1] * input_shape[2] * input_shape[3])
    loss = 0.25 * mse + mse
    out = jnp.transpose(zdl.reshape(input_shape), (0, 3, 1, 2))
    return out, loss, coeff


# speedup vs baseline: 10.2463x; 1.5054x over previous
import jax
import jax.numpy as jnp
from jax import lax
from jax.experimental import pallas as pl

_K = 512
_C = 64
_S = 5
_N = 16384
_BN = 256


def _bf(x):
    u = lax.bitcast_convert_type(x, jnp.int32)
    lsb = lax.shift_right_logical(u, 16) & 1
    u = u + (32767 + lsb)
    u = u & jnp.int32(-65536)
    return lax.bitcast_convert_type(u, jnp.float32)


def _prep_kernel(dn_ref, gram_ref, hi_ref, mid_ref, lo_ref):
    dn = dn_ref[...]
    gram = lax.dot_general(
        _bf(dn), _bf(dn), (((0,), (0,)), ((), ())),
        preferred_element_type=jnp.float32)
    gram_ref[...] = gram
    hi = _bf(gram)
    r = gram - hi
    mid = _bf(r)
    hi_ref[...] = hi
    mid_ref[...] = mid
    lo_ref[...] = r - mid


def _omp_kernel(s_ref, dn_ref, hi_ref, mid_ref, lo_ref, coeff_ref, zdl_ref, loss_ref):
    s = s_ref[...]
    dn = dn_ref[...]
    g_hi = hi_ref[...]
    g_mid = mid_ref[...]
    g_lo = lo_ref[...]
    init_corr = lax.dot_general(
        _bf(dn), _bf(s), (((0,), (0,)), ((), ())),
        preferred_element_type=jnp.float32)

    iota = lax.broadcasted_iota(jnp.int32, (_K, _BN), 0)
    corr = init_corr
    omega = jnp.ones((_K, _BN), jnp.float32)
    L = {}
    alphas = []
    g_cols = []
    sel_idx = []
    coeffs = None

    for kk in range(_S):
        a = jnp.abs(corr) * omega
        m = jnp.max(a, axis=0, keepdims=True)
        idx = jnp.min(jnp.where(a == m, iota, _K), axis=0, keepdims=True)
        onehot = (iota == idx).astype(jnp.float32)
        omega = omega * (1.0 - onehot)
        g = (jnp.dot(g_hi, onehot, preferred_element_type=jnp.float32)
             + jnp.dot(g_mid, onehot, preferred_element_type=jnp.float32)) \
            + jnp.dot(g_lo, onehot, preferred_element_type=jnp.float32)
        alphas.append(jnp.sum(init_corr * onehot, axis=0, keepdims=True))
        if kk > 0:
            gent = [jnp.sum(g_cols[j] * onehot, axis=0, keepdims=True)
                    for j in range(kk)]
            w = []
            for j in range(kk):
                acc = gent[j]
                for c in range(j):
                    acc = acc - L[(j, c)] * w[c]
                w.append(acc / L[(j, j)])
            wsq = w[0] * w[0]
            for j in range(1, kk):
                wsq = wsq + w[j] * w[j]
            for c in range(kk):
                L[(kk, c)] = w[c]
            L[(kk, kk)] = jnp.sqrt(1.0 - wsq)
        else:
            L[(0, 0)] = jnp.ones((1, _BN), jnp.float32)
        g_cols.append(g)
        sel_idx.append(idx)

        n = kk + 1
        mi = {}
        for i in range(n):
            mi[(i, i)] = 1.0 / L[(i, i)]
        for i in range(n):
            for j in range(i - 1, -1, -1):
                acc = jnp.zeros_like(L[(0, 0)])
                for mm in range(j, i):
                    acc = acc + L[(i, mm)] * mi[(mm, j)]
                mi[(i, j)] = -acc / L[(i, i)]
        y = []
        for i in range(n):
            acc = jnp.zeros_like(alphas[0])
            for j in range(i + 1):
                acc = acc + mi[(i, j)] * alphas[j]
            y.append(acc)
        cvec = [None] * n
        for i in range(n):
            acc = jnp.zeros_like(y[0])
            for j in range(i, n):
                acc = acc + mi[(j, i)] * y[j]
            cvec[i] = acc
        coeffs = cvec

        if kk < _S - 1:
            beta = cvec[0] * g_cols[0]
            for j in range(1, n):
                beta = beta + cvec[j] * g_cols[j]
            corr = init_corr - beta

    cd = coeffs[0] * (iota == sel_idx[0]).astype(jnp.float32)
    for j in range(1, _S):
        cd = cd + coeffs[j] * (iota == sel_idx[j]).astype(jnp.float32)
    coeff_ref[...] = cd

    zdl = lax.dot_general(_bf(dn), _bf(cd), (((1,), (0,)), ((), ())),
                          preferred_element_type=jnp.float32)
    zdl_ref[...] = zdl
    diff = zdl - s
    sq = jnp.sum(jnp.sum(diff * diff, axis=0, keepdims=True), axis=1, keepdims=True)

    i = pl.program_id(0)

    @pl.when(i == 0)
    def _init():
        loss_ref[...] = jnp.zeros((1, 1), jnp.float32)

    loss_ref[...] += sq


def kernel(z_e, dictionary):
    z = jnp.transpose(z_e, (0, 2, 3, 1))
    input_shape = z.shape
    s = z.reshape(_C, -1)
    dn = dictionary / jnp.linalg.norm(dictionary, axis=0)

    gram, g_hi, g_mid, g_lo = pl.pallas_call(
        _prep_kernel,
        out_shape=tuple(jax.ShapeDtypeStruct((_K, _K), jnp.float32)
                        for _ in range(4)),
    )(dn)

    grid = _N // _BN
    coeff, zdl, loss_sum = pl.pallas_call(
        _omp_kernel,
        grid=(grid,),
        in_specs=[
            pl.BlockSpec((_C, _BN), lambda i: (0, i)),
            pl.BlockSpec((_C, _K), lambda i: (0, 0)),
            pl.BlockSpec((_K, _K), lambda i: (0, 0)),
            pl.BlockSpec((_K, _K), lambda i: (0, 0)),
            pl.BlockSpec((_K, _K), lambda i: (0, 0)),
        ],
        out_specs=(
            pl.BlockSpec((_K, _BN), lambda i: (0, i)),
            pl.BlockSpec((_C, _BN), lambda i: (0, i)),
            pl.BlockSpec((1, 1), lambda i: (0, 0)),
        ),
        out_shape=(
            jax.ShapeDtypeStruct((_K, _N), jnp.float32),
            jax.ShapeDtypeStruct((_C, _N), jnp.float32),
            jax.ShapeDtypeStruct((1, 1), jnp.float32),
        ),
    )(s, dn, g_hi, g_mid, g_lo)

    mse = loss_sum[0, 0] / (input_shape[0] * input_shape[1] * input_shape[2] * input_shape[3])
    loss = 0.25 * mse + mse
    out = jnp.transpose(zdl.reshape(input_shape), (0, 3, 1, 2))
    return out, loss, coeff
